# baseline (speedup 1.0000x reference)
import functools

import jax
import jax.numpy as jnp
from jax import lax
from jax.experimental import pallas as pl
from jax.experimental.pallas import tpu as pltpu
from jax.experimental.pallas import tpu_sc as plsc

_G = 128
_C = 1024


def _gather_kernel(n_rows, embed_dim, num_workers):
    per_w = n_rows // num_workers
    n_chunks = per_w // _C
    n_g = _C // _G
    mesh = plsc.VectorSubcoreMesh(core_axis_name="c", subcore_axis_name="s")

    @functools.partial(
        pl.kernel,
        mesh=mesh,
        compiler_params=pltpu.CompilerParams(use_tc_tiling_on_sc=False),
        out_type=jax.ShapeDtypeStruct((n_rows, embed_dim), jnp.float32),
        scratch_types=[
            pltpu.VMEM((per_w // _G, _G), jnp.int32),
            pltpu.VMEM((_C, embed_dim), jnp.float32),
            pltpu.SemaphoreType.DMA,
        ],
    )
    def k(idx_hbm, tab_hbm, out_hbm, idx_v, rows_v, sem):
        wid = lax.axis_index("s") * 2 + lax.axis_index("c")
        row_base = wid * per_w
        pltpu.sync_copy(idx_hbm.at[pl.ds(wid * (per_w // _G), per_w // _G)], idx_v)

        def chunk(i, carry):
            handles = [
                pltpu.async_copy(
                    tab_hbm.at[idx_v.at[i * n_g + j]],
                    rows_v.at[pl.ds(j * _G, _G)],
                    sem,
                )
                for j in range(n_g)
            ]
            for h in handles:
                h.wait()
            pltpu.sync_copy(rows_v, out_hbm.at[pl.ds(row_base + i * _C, _C)])
            return carry

        lax.fori_loop(0, n_chunks, chunk, 0)

    return k


def kernel(batch_id, content):
    b, h = batch_id.shape
    v, d = content.shape
    n = b * h
    num_workers = 32
    idx2d = batch_id.reshape(n // _G, _G)
    out = _gather_kernel(n, d, num_workers)(idx2d, content)
    return out.reshape(b, h, d)


# speedup vs baseline: 1.0058x; 1.0058x over previous
import functools

import jax
import jax.numpy as jnp
from jax import lax
from jax.experimental import pallas as pl
from jax.experimental.pallas import tpu as pltpu
from jax.experimental.pallas import tpu_sc as plsc

_G = 128
_NB = 8


def _gather_kernel(n_rows, embed_dim, num_workers):
    per_w = n_rows // num_workers
    n_chunks = per_w // _G

    mesh = plsc.VectorSubcoreMesh(core_axis_name="c", subcore_axis_name="s")

    @functools.partial(
        pl.kernel,
        mesh=mesh,
        compiler_params=pltpu.CompilerParams(use_tc_tiling_on_sc=False),
        out_type=jax.ShapeDtypeStruct((n_rows, embed_dim), jnp.float32),
        scratch_types=[
            pltpu.VMEM((n_chunks, _G), jnp.int32),
            pltpu.VMEM((_NB * _G, embed_dim), jnp.float32),
        ]
        + [pltpu.SemaphoreType.DMA] * (2 * _NB),
    )
    def k(idx_hbm, tab_hbm, out_hbm, idx_v, rows_v, *sems):
        gsem, wsem = sems[:_NB], sems[_NB:]
        wid = lax.axis_index("s") * 2 + lax.axis_index("c")
        row_base = wid * per_w
        pltpu.sync_copy(idx_hbm.at[pl.ds(wid * n_chunks, n_chunks)], idx_v)

        def slot(b):
            return rows_v.at[pl.ds(b * _G, _G)]

        def fire_g(i, b):
            pltpu.async_copy(tab_hbm.at[idx_v.at[i]], slot(b), gsem[b])

        def wait_g(b):
            pltpu.make_async_copy(tab_hbm.at[pl.ds(0, _G)], slot(b), gsem[b]).wait()

        def fire_w(i, b):
            pltpu.async_copy(
                slot(b), out_hbm.at[pl.ds(row_base + i * _G, _G)], wsem[b]
            )

        def wait_w(b):
            pltpu.make_async_copy(
                slot(b), out_hbm.at[pl.ds(row_base, _G)], wsem[b]
            ).wait()

        for b in range(_NB - 1):
            fire_g(b, b)

        def body(g, carry):
            for b in range(_NB):
                i = g * _NB + b
                wait_g(b)
                fire_w(i, b)
                j = i + _NB - 1
                bj = (b - 1) % _NB

                @pl.when(j < n_chunks)
                def _():
                    @pl.when(j >= _NB)
                    def _():
                        wait_w(bj)

                    fire_g(j, bj)

            return carry

        lax.fori_loop(0, n_chunks // _NB, body, 0)
        for b in range(_NB):
            wait_w(b)

    return k


def kernel(batch_id, content):
    b, h = batch_id.shape
    v, d = content.shape
    n = b * h
    num_workers = 32
    idx2d = batch_id.reshape(n // _G, _G)
    out = _gather_kernel(n, d, num_workers)(idx2d, content)
    return out.reshape(b, h, d)
